# baseline (speedup 1.0000x reference)
import functools

import jax
import jax.numpy as jnp
from jax import lax
from jax.experimental import pallas as pl
from jax.experimental.pallas import tpu as pltpu
from jax.experimental.pallas import tpu_sc as plsc

N = 10000
E = 320000
F = 128
NC = 2
NS = 16
NW = NC * NS
NPAD = 10240
ST = NPAD // NS
C = 80
EPW = E // NW
KPW = EPW // C
EPT = E // NS
KPT = EPT // C


def _sc_body(row_h, col_h, ew_h, x_h, out_h,
             dis_v, degb, normb,
             cs0, cs1, rs0, rs1, ws0, ws1, xbuf0, xbuf1,
             deg_sh, dis_sh, s_sh, semg0, semg1, semp0, semp1):
    cid = lax.axis_index("c")
    sid = lax.axis_index("s")
    wid = cid * NS + sid
    xbufs = (xbuf0, xbuf1)
    semgs = (semg0, semg1)
    css = (cs0, cs1)
    rss = (rs0, rs1)
    wss = (ws0, ws1)
    semps = (semp0, semp1)

    scope = jax.named_scope
    def zx(j, _):
        for f in range(F // 16):
            xbuf0[j, pl.ds(f * 16, 16)] = jnp.zeros((16,), jnp.float32)
        return 0
    lax.fori_loop(0, C, zx, 0)

    def zd(g, _):
        degb[pl.ds(g * 16, 16)] = jnp.zeros((16,), jnp.float32)
        return 0
    lax.fori_loop(0, ST // 16, zd, 0)

    pltpu.sync_copy(degb, deg_sh.at[pl.ds(sid * ST, ST)])
    for i in range(ST // C):
        pltpu.sync_copy(xbuf0, s_sh.at[pl.ds(sid * ST + i * C, C)])
    plsc.subcore_barrier()

    dbase = sid * EPT

    def dpre(k, p):
        pltpu.async_copy(col_h.at[pl.ds(dbase + k * C, C)], css[p], semps[p])
        pltpu.async_copy(ew_h.at[pl.ds(dbase + k * C, C)], wss[p], semps[p])

    def dwait(k, p):
        pltpu.make_async_copy(col_h.at[pl.ds(dbase + k * C, C)], css[p],
                              semps[p]).wait()
        pltpu.make_async_copy(ew_h.at[pl.ds(dbase + k * C, C)], wss[p],
                              semps[p]).wait()

    dpre(0, 0)

    def dstep(i, _):
        for p in range(2):
            k = 2 * i + p
            dwait(k, p)

            @pl.when(k + 1 <= KPT - 1)
            def _pre():
                dpre(k + 1, 1 - p)
            pltpu.sync_copy(wss[p], deg_sh.at[css[p]], add=True)
        return 0
    with scope("ph_deg"):
        lax.fori_loop(0, KPT // 2, dstep, 0)
    plsc.subcore_barrier()

    pltpu.sync_copy(deg_sh.at[pl.ds(sid * ST, ST)], degb)

    def dis_step(g, _):
        d = degb[pl.ds(g * 16, 16)]
        i = lax.bitcast_convert_type(d, jnp.int32)
        i = jnp.int32(0x5F3759DF) - lax.shift_right_arithmetic(i, 1)
        y = lax.bitcast_convert_type(i, jnp.float32)
        for _ in range(3):
            y = y * (1.5 - 0.5 * d * y * y)
        degb[pl.ds(g * 16, 16)] = jnp.where(d > 0.0, y, 0.0)
        return 0
    with scope("ph_dis"):
        lax.fori_loop(0, ST // 16, dis_step, 0)
    pltpu.sync_copy(degb, dis_sh.at[pl.ds(sid * ST, ST)])
    plsc.subcore_barrier()
    pltpu.sync_copy(dis_sh, dis_v)

    base = wid * EPW

    def npre(k, p):
        off = base + k * C
        pltpu.async_copy(row_h.at[pl.ds(off, C)], rss[p], semps[p])
        pltpu.async_copy(col_h.at[pl.ds(off, C)], css[p], semps[p])
        pltpu.async_copy(ew_h.at[pl.ds(off, C)], wss[p], semps[p])

    def nwait(k, p):
        off = base + k * C
        pltpu.make_async_copy(row_h.at[pl.ds(off, C)], rss[p],
                              semps[p]).wait()
        pltpu.make_async_copy(col_h.at[pl.ds(off, C)], css[p],
                              semps[p]).wait()
        pltpu.make_async_copy(ew_h.at[pl.ds(off, C)], wss[p],
                              semps[p]).wait()

    def ncompute(k, p):
        for g in range(C // 16):
            r16 = rss[p][pl.ds(g * 16, 16)]
            c16 = css[p][pl.ds(g * 16, 16)]
            w16 = wss[p][pl.ds(g * 16, 16)]
            dr = plsc.load_gather(dis_v, [r16])
            dc = plsc.load_gather(dis_v, [c16])
            normb[pl.ds(k * C + g * 16, 16)] = dr * w16 * dc

    npre(0, 0)

    def nstep(i, _):
        for p in range(2):
            k = 2 * i + p
            nwait(k, p)

            @pl.when(k + 1 <= KPW - 1)
            def _pre():
                npre(k + 1, 1 - p)
            ncompute(k, p)
        return 0
    with scope("ph_norm"):
        lax.fori_loop(0, KPW // 2, nstep, 0)
        nwait(KPW - 1, 0)
        ncompute(KPW - 1, 0)

    def process(q, b):
        def scale(u, _):
            for t in range(2):
                jj = 2 * u + t
                nj = plsc.load_gather(
                    normb, [jnp.full((16,), q * C + jj, jnp.int32)])
                for f in range(F // 16):
                    xbufs[b][jj, pl.ds(f * 16, 16)] = (
                        xbufs[b][jj, pl.ds(f * 16, 16)] * nj)
            return 0
        lax.fori_loop(0, C // 2, scale, 0)
        pltpu.sync_copy(xbufs[b], s_sh.at[css[b]], add=True)

    def ipre(q, b):
        off = base + q * C
        pltpu.async_copy(row_h.at[pl.ds(off, C)], rss[b], semps[b])
        pltpu.async_copy(col_h.at[pl.ds(off, C)], css[b], semps[b])

    def iwait(q, b):
        off = base + q * C
        pltpu.make_async_copy(row_h.at[pl.ds(off, C)], rss[b],
                              semps[b]).wait()
        pltpu.make_async_copy(col_h.at[pl.ds(off, C)], css[b],
                              semps[b]).wait()

    ipre(0, 0)
    iwait(0, 0)
    pltpu.async_copy(x_h.at[rs0], xbuf0, semg0)
    ipre(1, 1)

    def mbody(i, _):
        for j in range(2):
            q = 2 * i + j
            pltpu.make_async_copy(x_h.at[rss[j]], xbufs[j], semgs[j]).wait()
            iwait(q + 1, 1 - j)
            pltpu.async_copy(x_h.at[rss[1 - j]], xbufs[1 - j],
                             semgs[1 - j])
            process(q, j)

            @pl.when(q + 2 <= KPW - 1)
            def _pre():
                ipre(q + 2, j)
        return 0
    with scope("ph_main"):
        lax.fori_loop(0, (KPW - 1) // 2, mbody, 0)
    pltpu.make_async_copy(x_h.at[rs0], xbuf0, semg0).wait()
    process(KPW - 1, 0)
    plsc.subcore_barrier()

    with scope("ph_wout"):
        pltpu.sync_copy(s_sh.at[pl.ds(sid * ST, ST)],
                        out_h.at[cid, pl.ds(sid * ST, ST)])


_sc_scatter = pl.kernel(
    _sc_body,
    out_type=jax.ShapeDtypeStruct((NC, NPAD, F), jnp.float32),
    mesh=plsc.VectorSubcoreMesh(core_axis_name="c", subcore_axis_name="s",
                                num_cores=NC, num_subcores=NS),
    scratch_types=[
        pltpu.VMEM((NPAD,), jnp.float32),
        pltpu.VMEM((ST,), jnp.float32),
        pltpu.VMEM((EPW,), jnp.float32),
        pltpu.VMEM((C,), jnp.int32),
        pltpu.VMEM((C,), jnp.int32),
        pltpu.VMEM((C,), jnp.int32),
        pltpu.VMEM((C,), jnp.int32),
        pltpu.VMEM((C,), jnp.float32),
        pltpu.VMEM((C,), jnp.float32),
        pltpu.VMEM((C, F), jnp.float32),
        pltpu.VMEM((C, F), jnp.float32),
        pltpu.VMEM_SHARED((NPAD,), jnp.float32),
        pltpu.VMEM_SHARED((NPAD,), jnp.float32),
        pltpu.VMEM_SHARED((NPAD, F), jnp.float32),
        pltpu.SemaphoreType.DMA,
        pltpu.SemaphoreType.DMA,
        pltpu.SemaphoreType.DMA,
        pltpu.SemaphoreType.DMA,
    ],
    compiler_params=pltpu.CompilerParams(needs_layout_passes=False),
)


def _tc_body(x_ref, s0_ref, s1_ref, wl_ref, wg_ref, o_ref):
    dn = (((1,), (1,)), ((), ()))
    s = s0_ref[...] + s1_ref[...]
    o_ref[...] = (
        lax.dot_general(x_ref[...], wl_ref[...], dn,
                        preferred_element_type=jnp.float32,
                        precision=lax.Precision.HIGHEST)
        + lax.dot_general(s, wg_ref[...], dn,
                          preferred_element_type=jnp.float32,
                          precision=lax.Precision.HIGHEST))


_BLK = 1000


def _tc_combine(x, s0, s1, W_lin, W_gcn):
    grid = (N // _BLK,)
    row_spec = pl.BlockSpec((_BLK, F), lambda i: (i, 0))
    w_spec = pl.BlockSpec((F, F), lambda i: (0, 0))
    return pl.pallas_call(
        _tc_body,
        grid=grid,
        in_specs=[row_spec, row_spec, row_spec, w_spec, w_spec],
        out_specs=row_spec,
        out_shape=jax.ShapeDtypeStruct((N, F), jnp.float32),
    )(x, s0, s1, W_lin, W_gcn)


@jax.jit
def kernel(x, edge_index, edge_weight, W_lin, W_gcn):
    row = edge_index[0]
    col = edge_index[1]
    s_part = _sc_scatter(row, col, edge_weight, x)
    return _tc_combine(x, s_part[0, :N], s_part[1, :N], W_lin, W_gcn)


# speedup vs baseline: 1.3027x; 1.3027x over previous
import functools

import jax
import jax.numpy as jnp
from jax import lax
from jax.experimental import pallas as pl
from jax.experimental.pallas import tpu as pltpu
from jax.experimental.pallas import tpu_sc as plsc

N = 10000
E = 320000
F = 128
NC = 2
NS = 16
NW = NC * NS
NPAD = 10240
ST = NPAD // NS
C = 80
EPW = E // NW
KPW = EPW // C
EPT = E // NS
KPT = EPT // C


def _sc_body(row_h, col_h, ew_h, x_h, out_h,
             dis_v, degb, normv,
             rs0, rs1, rs2, cs0, cs1, cs2, ws0, ws1, ws2,
             cd0, cd1, cd2, cd3, wd0, wd1, wd2, wd3,
             xbuf0, xbuf1, xbuf2,
             deg_sh, dis_sh, s_sh,
             semg0, semg1, semg2, semp0, semp1, semp2,
             semd0, semd1, semd2, semd3, sempd0, sempd1, sempd2, sempd3):
    cid = lax.axis_index("c")
    sid = lax.axis_index("s")
    wid = cid * NS + sid
    xbufs = (xbuf0, xbuf1, xbuf2)
    semgs = (semg0, semg1, semg2)
    rss = (rs0, rs1, rs2)
    css = (cs0, cs1, cs2)
    wss = (ws0, ws1, ws2)
    semps = (semp0, semp1, semp2)
    cds = (cd0, cd1, cd2, cd3)
    wds = (wd0, wd1, wd2, wd3)
    semds = (semd0, semd1, semd2, semd3)
    sempds = (sempd0, sempd1, sempd2, sempd3)

    def zx(j, _):
        for f in range(F // 16):
            xbuf0[j, pl.ds(f * 16, 16)] = jnp.zeros((16,), jnp.float32)
        return 0
    lax.fori_loop(0, C, zx, 0)

    def zd(g, _):
        degb[pl.ds(g * 16, 16)] = jnp.zeros((16,), jnp.float32)
        return 0
    lax.fori_loop(0, ST // 16, zd, 0)

    pltpu.sync_copy(degb, deg_sh.at[pl.ds(sid * ST, ST)])
    for i in range(ST // C):
        pltpu.sync_copy(xbuf0, s_sh.at[pl.ds(sid * ST + i * C, C)])
    plsc.subcore_barrier()

    dbase = sid * EPT

    def dpre(k, s):
        pltpu.async_copy(col_h.at[pl.ds(dbase + k * C, C)], cds[s],
                         sempds[s])
        pltpu.async_copy(ew_h.at[pl.ds(dbase + k * C, C)], wds[s],
                         sempds[s])

    def dwait(k, s):
        pltpu.make_async_copy(col_h.at[pl.ds(dbase + k * C, C)], cds[s],
                              sempds[s]).wait()
        pltpu.make_async_copy(ew_h.at[pl.ds(dbase + k * C, C)], wds[s],
                              sempds[s]).wait()

    def dscat(s):
        pltpu.async_copy(wds[s], deg_sh.at[cds[s]], semds[s], add=True)

    def dscat_wait(s):
        pltpu.make_async_copy(wds[s], deg_sh.at[cds[s]], semds[s]).wait()

    dpre(0, 0)
    dpre(1, 1)
    dwait(0, 0)
    dpre(2, 2)
    dscat(0)
    dwait(1, 1)
    dpre(3, 3)
    dscat(1)

    def dstep(i, _):
        for j in range(4):
            k = 2 + 4 * i + j
            s = (2 + j) % 4
            dwait(k, s)
            dscat_wait(j)

            @pl.when(k + 2 <= KPT - 1)
            def _pre():
                dpre(k + 2, j)
            dscat(s)
        return 0
    lax.fori_loop(0, (KPT - 2) // 4, dstep, 0)
    dscat_wait(0)
    dscat_wait(1)
    plsc.subcore_barrier()

    pltpu.sync_copy(deg_sh.at[pl.ds(sid * ST, ST)], degb)

    def dis_step(g, _):
        d = degb[pl.ds(g * 16, 16)]
        i = lax.bitcast_convert_type(d, jnp.int32)
        i = jnp.int32(0x5F3759DF) - lax.shift_right_arithmetic(i, 1)
        y = lax.bitcast_convert_type(i, jnp.float32)
        for _ in range(3):
            y = y * (1.5 - 0.5 * d * y * y)
        degb[pl.ds(g * 16, 16)] = jnp.where(d > 0.0, y, 0.0)
        return 0
    lax.fori_loop(0, ST // 16, dis_step, 0)
    pltpu.sync_copy(degb, dis_sh.at[pl.ds(sid * ST, ST)])
    plsc.subcore_barrier()
    pltpu.sync_copy(dis_sh, dis_v)

    base = wid * EPW

    def ipre(q, b):
        off = base + q * C
        pltpu.async_copy(row_h.at[pl.ds(off, C)], rss[b], semps[b])
        pltpu.async_copy(col_h.at[pl.ds(off, C)], css[b], semps[b])
        pltpu.async_copy(ew_h.at[pl.ds(off, C)], wss[b], semps[b])

    def iwait(q, b):
        off = base + q * C
        pltpu.make_async_copy(row_h.at[pl.ds(off, C)], rss[b],
                              semps[b]).wait()
        pltpu.make_async_copy(col_h.at[pl.ds(off, C)], css[b],
                              semps[b]).wait()
        pltpu.make_async_copy(ew_h.at[pl.ds(off, C)], wss[b],
                              semps[b]).wait()

    def process(q, b):
        for g in range(C // 16):
            r16 = rss[b][pl.ds(g * 16, 16)]
            c16 = css[b][pl.ds(g * 16, 16)]
            w16 = wss[b][pl.ds(g * 16, 16)]
            dr = plsc.load_gather(dis_v, [r16])
            dc = plsc.load_gather(dis_v, [c16])
            normv[pl.ds(g * 16, 16)] = dr * w16 * dc

        def scale(u, _):
            for t in range(2):
                jj = 2 * u + t
                nj = plsc.load_gather(normv, [jnp.full((16,), jj, jnp.int32)])
                for f in range(F // 16):
                    xbufs[b][jj, pl.ds(f * 16, 16)] = (
                        xbufs[b][jj, pl.ds(f * 16, 16)] * nj)
            return 0
        lax.fori_loop(0, C // 2, scale, 0)
        pltpu.sync_copy(xbufs[b], s_sh.at[css[b]], add=True)

    def T(q, b):
        pltpu.make_async_copy(x_h.at[rss[b]], xbufs[b], semgs[b]).wait()
        b2 = (b + 2) % 3

        @pl.when(q + 2 <= KPW - 1)
        def _g():
            iwait(q + 2, b2)
            pltpu.async_copy(x_h.at[rss[b2]], xbufs[b2], semgs[b2])
        process(q, b)

        @pl.when(q + 3 <= KPW - 1)
        def _p():
            ipre(q + 3, b)

    ipre(0, 0)
    ipre(1, 1)
    ipre(2, 2)
    iwait(0, 0)
    pltpu.async_copy(x_h.at[rs0], xbuf0, semg0)
    iwait(1, 1)
    pltpu.async_copy(x_h.at[rs1], xbuf1, semg1)
    T(0, 0)
    T(1, 1)

    def mbody(i, _):
        for j in range(3):
            q = 2 + 3 * i + j
            T(q, (2 + j) % 3)
        return 0
    lax.fori_loop(0, (KPW - 2) // 3, mbody, 0)
    plsc.subcore_barrier()

    pltpu.sync_copy(s_sh.at[pl.ds(sid * ST, ST)],
                    out_h.at[cid, pl.ds(sid * ST, ST)])


_sc_scatter = pl.kernel(
    _sc_body,
    out_type=jax.ShapeDtypeStruct((NC, NPAD, F), jnp.float32),
    mesh=plsc.VectorSubcoreMesh(core_axis_name="c", subcore_axis_name="s",
                                num_cores=NC, num_subcores=NS),
    scratch_types=[
        pltpu.VMEM((NPAD,), jnp.float32),
        pltpu.VMEM((ST,), jnp.float32),
        pltpu.VMEM((C,), jnp.float32),
        pltpu.VMEM((C,), jnp.int32),
        pltpu.VMEM((C,), jnp.int32),
        pltpu.VMEM((C,), jnp.int32),
        pltpu.VMEM((C,), jnp.int32),
        pltpu.VMEM((C,), jnp.int32),
        pltpu.VMEM((C,), jnp.int32),
        pltpu.VMEM((C,), jnp.float32),
        pltpu.VMEM((C,), jnp.float32),
        pltpu.VMEM((C,), jnp.float32),
        pltpu.VMEM((C,), jnp.int32),
        pltpu.VMEM((C,), jnp.int32),
        pltpu.VMEM((C,), jnp.int32),
        pltpu.VMEM((C,), jnp.int32),
        pltpu.VMEM((C,), jnp.float32),
        pltpu.VMEM((C,), jnp.float32),
        pltpu.VMEM((C,), jnp.float32),
        pltpu.VMEM((C,), jnp.float32),
        pltpu.VMEM((C, F), jnp.float32),
        pltpu.VMEM((C, F), jnp.float32),
        pltpu.VMEM((C, F), jnp.float32),
        pltpu.VMEM_SHARED((NPAD,), jnp.float32),
        pltpu.VMEM_SHARED((NPAD,), jnp.float32),
        pltpu.VMEM_SHARED((NPAD, F), jnp.float32),
        pltpu.SemaphoreType.DMA,
        pltpu.SemaphoreType.DMA,
        pltpu.SemaphoreType.DMA,
        pltpu.SemaphoreType.DMA,
        pltpu.SemaphoreType.DMA,
        pltpu.SemaphoreType.DMA,
        pltpu.SemaphoreType.DMA,
        pltpu.SemaphoreType.DMA,
        pltpu.SemaphoreType.DMA,
        pltpu.SemaphoreType.DMA,
        pltpu.SemaphoreType.DMA,
        pltpu.SemaphoreType.DMA,
        pltpu.SemaphoreType.DMA,
        pltpu.SemaphoreType.DMA,
    ],
    compiler_params=pltpu.CompilerParams(needs_layout_passes=False),
)


def _tc_body(x_ref, s0_ref, s1_ref, wl_ref, wg_ref, o_ref):
    dn = (((1,), (1,)), ((), ()))
    s = s0_ref[...] + s1_ref[...]
    o_ref[...] = (
        lax.dot_general(x_ref[...], wl_ref[...], dn,
                        preferred_element_type=jnp.float32,
                        precision=lax.Precision.HIGHEST)
        + lax.dot_general(s, wg_ref[...], dn,
                          preferred_element_type=jnp.float32,
                          precision=lax.Precision.HIGHEST))


_BLK = 1000


def _tc_combine(x, s0, s1, W_lin, W_gcn):
    grid = (N // _BLK,)
    row_spec = pl.BlockSpec((_BLK, F), lambda i: (i, 0))
    w_spec = pl.BlockSpec((F, F), lambda i: (0, 0))
    return pl.pallas_call(
        _tc_body,
        grid=grid,
        in_specs=[row_spec, row_spec, row_spec, w_spec, w_spec],
        out_specs=row_spec,
        out_shape=jax.ShapeDtypeStruct((N, F), jnp.float32),
    )(x, s0, s1, W_lin, W_gcn)


@jax.jit
def kernel(x, edge_index, edge_weight, W_lin, W_gcn):
    row = edge_index[0]
    col = edge_index[1]
    s_part = _sc_scatter(row, col, edge_weight, x)
    return _tc_combine(x, s_part[0, :N], s_part[1, :N], W_lin, W_gcn)
